# baseline (speedup 1.0000x reference)
import functools

import jax
import jax.numpy as jnp
from jax import lax
from jax.experimental import pallas as pl
from jax.experimental.pallas import tpu as pltpu
from jax.experimental.pallas import tpu_sc as plsc

_B, _N, _D = 4, 1024, 320
_DL, _H, _K = 256, 256, 8
_R1 = 256
_R3 = 256
_NC, _NS = 2, 16
_NW = _NC * _NS
_GCHUNK = 128
_CW = 640


def _dot_t(x, w):
    return lax.dot_general(x, w, (((1,), (1,)), ((), ())),
                           preferred_element_type=jnp.float32)


def _stage1_body(cenall_ref, of_ref, siz_ref, lang_ref,
                 w1a_ref, w1b_ref, wgc_ref, wgs_ref, w1l_ref, b1_ref,
                 idx_ref, gidxt_ref, a_ref, c_ref):
    b = pl.program_id(0)
    i = pl.program_id(1)
    cen = cenall_ref[0, pl.ds(i * _R1, _R1), :]
    cenT = jnp.transpose(cenall_ref[0], (1, 0))
    d = jnp.zeros((_R1, _N), jnp.float32)
    for c in range(3):
        diff = cen[:, c:c + 1] - cenT[c:c + 1, :]
        d = d + diff * diff
    d = d / 25.0
    rowg = i * _R1 + lax.broadcasted_iota(jnp.int32, (_R1, _N), 0)
    colid_i = lax.broadcasted_iota(jnp.int32, (_R1, _N), 1)
    d = jnp.where(rowg == colid_i, jnp.inf, d)
    colid = colid_i.astype(jnp.float32)
    sels = []
    for k in range(_K):
        m = jnp.min(d, axis=1, keepdims=True)
        sel = jnp.min(jnp.where(d == m, colid, jnp.float32(_N)),
                      axis=1, keepdims=True)
        sels.append(sel)
        d = jnp.where(colid == sel, jnp.inf, d)
    idxf = jnp.concatenate(sels, axis=1)
    idx_ref[0] = idxf.astype(jnp.int32)
    gidxt_ref[...] = (jnp.transpose(idxf, (1, 0))
                      + jnp.float32(b * _N)).astype(jnp.int32)

    of = of_ref[0]
    g = (_dot_t(cen, wgc_ref[...]) * (1.0 / (5.0 + 1e-06))
         + _dot_t(siz_ref[0], wgs_ref[...]) * 0.5)
    langp = _dot_t(lang_ref[0], w1l_ref[...])
    a_ref[0] = _dot_t(of, w1a_ref[...]) + langp + b1_ref[...] + g
    cvals = _dot_t(of, w1b_ref[...]) - g
    u = lax.bitcast_convert_type(cvals, jnp.uint32)
    rb = (u + jnp.uint32(0x7FFF) + ((u >> 16) & jnp.uint32(1))) >> 16
    c_ref[0] = (rb[:, :_H // 2] | (rb[:, _H // 2:] << 16)).astype(jnp.uint32)


def _stage1(cen, of, siz, lang, w1a, w1b, wgc, wgs, w1l, b1):
    grid = (_B, _N // _R1)
    return pl.pallas_call(
        _stage1_body,
        grid=grid,
        in_specs=[
            pl.BlockSpec((1, _N, 3), lambda b, i: (b, 0, 0)),
            pl.BlockSpec((1, _R1, _D), lambda b, i: (b, i, 0)),
            pl.BlockSpec((1, _R1, 3), lambda b, i: (b, i, 0)),
            pl.BlockSpec((1, 1, _DL), lambda b, i: (b, 0, 0)),
            pl.BlockSpec((_H, _D), lambda b, i: (0, 0)),
            pl.BlockSpec((_H, _D), lambda b, i: (0, 0)),
            pl.BlockSpec((_H, 3), lambda b, i: (0, 0)),
            pl.BlockSpec((_H, 3), lambda b, i: (0, 0)),
            pl.BlockSpec((_H, _DL), lambda b, i: (0, 0)),
            pl.BlockSpec((1, _H), lambda b, i: (0, 0)),
        ],
        out_specs=[
            pl.BlockSpec((1, _R1, _K), lambda b, i: (b, i, 0)),
            pl.BlockSpec((_K, _R1), lambda b, i: (0, b * (_N // _R1) + i)),
            pl.BlockSpec((1, _R1, _H), lambda b, i: (b, i, 0)),
            pl.BlockSpec((1, _R1, _H // 2), lambda b, i: (b, i, 0)),
        ],
        out_shape=[
            jax.ShapeDtypeStruct((_B, _N, _K), jnp.int32),
            jax.ShapeDtypeStruct((_K, _B * _N), jnp.int32),
            jax.ShapeDtypeStruct((_B, _N, _H), jnp.float32),
            jax.ShapeDtypeStruct((_B, _N, _H // 2), jnp.uint32),
        ],
    )(cen, of, siz, lang, w1a, w1b, wgc, wgs, w1l, b1)


def _sc_gather(gidxt, c_all, nb):
    n_idx = nb * _N * _K
    per_w = n_idx // _NW
    n_chunks = max(per_w // _GCHUNK, 1)
    chunk = min(per_w, _GCHUNK)
    span = nb * _N
    wpk = _NW // _K

    mesh = plsc.VectorSubcoreMesh(core_axis_name="c", subcore_axis_name="s")

    @functools.partial(
        pl.kernel,
        mesh=mesh,
        out_type=jax.ShapeDtypeStruct((n_idx, _H // 2), jnp.uint32),
        scratch_types=[
            pltpu.VMEM((per_w,), jnp.int32),
            pltpu.VMEM((2, _GCHUNK, _H // 2), jnp.uint32),
            pltpu.SemaphoreType.DMA,
        ],
    )
    def k(gidx_hbm, c_hbm, out_hbm, idx_v, rows_v, gsem):
        wid = lax.axis_index("s") * _NC + lax.axis_index("c")
        base0 = wid * per_w
        pltpu.sync_copy(
            gidx_hbm.at[wid // wpk, pl.ds((wid % wpk) * per_w, per_w)], idx_v)
        cps = [None, None]
        cps[0] = pltpu.async_copy(
            c_hbm.at[idx_v.at[pl.ds(0, chunk)]], rows_v.at[0], gsem)
        for t in range(n_chunks):
            s = t % 2
            if t + 1 < n_chunks:
                cps[1 - s] = pltpu.async_copy(
                    c_hbm.at[idx_v.at[pl.ds((t + 1) * chunk, chunk)]],
                    rows_v.at[1 - s], gsem)
            cps[s].wait()
            pltpu.sync_copy(
                rows_v.at[s], out_hbm.at[pl.ds(base0 + t * chunk, chunk)])

    return k(gidxt, c_all)


def _stage3_body(a_ref, cg_ref, idx_ref, offull_ref, w2_ref, b2_ref,
                 out_ref, w_ref):
    i = pl.program_id(1)
    a = a_ref[0]
    cols = []
    for k in range(_K):
        pk = cg_ref[k]
        lo = lax.bitcast_convert_type(pk << 16, jnp.float32)
        hi = lax.bitcast_convert_type(pk & jnp.uint32(0xFFFF0000), jnp.float32)
        cgk = jnp.concatenate([lo, hi], axis=1)
        h = jnp.maximum(a + cgk, 0.0)
        cols.append(_dot_t(h, w2_ref[...]))
    scores = jnp.concatenate(cols, axis=1) + b2_ref[...]
    m = jnp.max(scores, axis=1, keepdims=True)
    e = jnp.exp(scores - m)
    w = e / jnp.sum(e, axis=1, keepdims=True)
    idx = idx_ref[0]
    colid = lax.broadcasted_iota(jnp.int32, (_R3, _N), 1)
    wmat = jnp.zeros((_R3, _N), jnp.float32)
    for k in range(_K):
        wmat = jnp.where(colid == idx[:, k:k + 1], w[:, k:k + 1], wmat)
    ctx = jnp.dot(wmat, offull_ref[0], preferred_element_type=jnp.float32)
    out_ref[0] = offull_ref[0, pl.ds(i * _R3, _R3), :] + ctx
    w_ref[0] = w


def _stage3(a_all, cg3, idx, of, w2, b2, nb=_B):
    grid = (nb, _N // _R3)
    return pl.pallas_call(
        _stage3_body,
        grid=grid,
        in_specs=[
            pl.BlockSpec((1, _R3, _H), lambda b, i: (b, i, 0)),
            pl.BlockSpec((_K, _R3, _H // 2),
                         lambda b, i: (0, b * (_N // _R3) + i, 0)),
            pl.BlockSpec((1, _R3, _K), lambda b, i: (b, i, 0)),
            pl.BlockSpec((1, _N, _D), lambda b, i: (b, 0, 0)),
            pl.BlockSpec((1, _H), lambda b, i: (0, 0)),
            pl.BlockSpec((1, _K), lambda b, i: (0, 0)),
        ],
        out_specs=[
            pl.BlockSpec((1, _R3, _D), lambda b, i: (b, i, 0)),
            pl.BlockSpec((1, _R3, _K), lambda b, i: (b, i, 0)),
        ],
        out_shape=[
            jax.ShapeDtypeStruct((nb, _N, _D), jnp.float32),
            jax.ShapeDtypeStruct((nb, _N, _K), jnp.float32),
        ],
    )(a_all, cg3, idx, of, w2, b2)


def kernel(object_features, language_embedding, centers, sizes, W1, b1, W2, b2):
    w1a = W1[:, :_D]
    w1b = W1[:, _D:2 * _D]
    wgc = W1[:, 2 * _D:2 * _D + 3]
    wgs = W1[:, 2 * _D + 3:2 * _D + 6]
    w1l = W1[:, 2 * _D + 6:]
    b1r = b1.reshape(1, _H)
    b2r = jnp.broadcast_to(b2.reshape(1, 1), (1, _K))

    idx, gidxt, a_all, c_all = _stage1(
        centers, object_features, sizes,
        language_embedding.reshape(_B, 1, _DL),
        w1a, w1b, wgc, wgs, w1l, b1r)

    c_tab = c_all.reshape(_B * _N, _H // 2)
    outs, ws = [], []
    for b in range(_B):
        cg_b = _sc_gather(gidxt[:, b * _N:(b + 1) * _N], c_tab, 1)
        o_b, w_b = _stage3(
            a_all[b:b + 1], cg_b.reshape(_K, _N, _H // 2), idx[b:b + 1],
            object_features[b:b + 1], W2, b2r, 1)
        outs.append(o_b)
        ws.append(w_b)

    return (jnp.concatenate(outs, axis=0), jnp.concatenate(ws, axis=0), idx)


# speedup vs baseline: 1.2222x; 1.2222x over previous
import functools

import jax
import jax.numpy as jnp
from jax import lax
from jax.experimental import pallas as pl
from jax.experimental.pallas import tpu as pltpu
from jax.experimental.pallas import tpu_sc as plsc

_B, _N, _D = 4, 1024, 320
_DL, _H, _K = 256, 256, 8
_R1 = 512
_R3 = 512
_NC, _NS = 2, 16
_NW = _NC * _NS
_GCHUNK = 128
_CW = 640


def _dot_t(x, w):
    return lax.dot_general(x, w, (((1,), (1,)), ((), ())),
                           preferred_element_type=jnp.float32)


def _stage1_body(cenall_ref, of_ref, siz_ref, lang_ref,
                 w1a_ref, w1b_ref, wgc_ref, wgs_ref, w1l_ref, b1_ref,
                 idx_ref, gidxt_ref, a_ref, c_ref):
    b = pl.program_id(0)
    i = pl.program_id(1)
    cen = cenall_ref[0, pl.ds(i * _R1, _R1), :]
    cenT = jnp.transpose(cenall_ref[0], (1, 0))
    d = jnp.zeros((_R1, _N), jnp.float32)
    for c in range(3):
        diff = cen[:, c:c + 1] - cenT[c:c + 1, :]
        d = d + diff * diff
    d = d / 25.0
    rowg = i * _R1 + lax.broadcasted_iota(jnp.int32, (_R1, _N), 0)
    colid_i = lax.broadcasted_iota(jnp.int32, (_R1, _N), 1)
    d = jnp.where(rowg == colid_i, jnp.inf, d)
    colid = colid_i.astype(jnp.float32)
    sels = []
    for k in range(_K):
        m = jnp.min(d, axis=1, keepdims=True)
        sel = jnp.min(jnp.where(d == m, colid, jnp.float32(_N)),
                      axis=1, keepdims=True)
        sels.append(sel)
        d = jnp.where(colid == sel, jnp.inf, d)
    idxf = jnp.concatenate(sels, axis=1)
    idx_ref[0] = idxf.astype(jnp.int32)
    gidxt_ref[...] = (jnp.transpose(idxf, (1, 0))
                      + jnp.float32(b * _N)).astype(jnp.int32)

    of = of_ref[0]
    g = (_dot_t(cen, wgc_ref[...]) * (1.0 / (5.0 + 1e-06))
         + _dot_t(siz_ref[0], wgs_ref[...]) * 0.5)
    langp = _dot_t(lang_ref[0], w1l_ref[...])
    a_ref[0] = _dot_t(of, w1a_ref[...]) + langp + b1_ref[...] + g
    cvals = _dot_t(of, w1b_ref[...]) - g
    u = lax.bitcast_convert_type(cvals, jnp.uint32)
    rb = (u + jnp.uint32(0x7FFF) + ((u >> 16) & jnp.uint32(1))) >> 16
    c_ref[0] = (rb[:, :_H // 2] | (rb[:, _H // 2:] << 16)).astype(jnp.uint32)


def _stage1(cen, of, siz, lang, w1a, w1b, wgc, wgs, w1l, b1):
    grid = (_B, _N // _R1)
    return pl.pallas_call(
        _stage1_body,
        grid=grid,
        in_specs=[
            pl.BlockSpec((1, _N, 3), lambda b, i: (b, 0, 0)),
            pl.BlockSpec((1, _R1, _D), lambda b, i: (b, i, 0)),
            pl.BlockSpec((1, _R1, 3), lambda b, i: (b, i, 0)),
            pl.BlockSpec((1, 1, _DL), lambda b, i: (b, 0, 0)),
            pl.BlockSpec((_H, _D), lambda b, i: (0, 0)),
            pl.BlockSpec((_H, _D), lambda b, i: (0, 0)),
            pl.BlockSpec((_H, 3), lambda b, i: (0, 0)),
            pl.BlockSpec((_H, 3), lambda b, i: (0, 0)),
            pl.BlockSpec((_H, _DL), lambda b, i: (0, 0)),
            pl.BlockSpec((1, _H), lambda b, i: (0, 0)),
        ],
        out_specs=[
            pl.BlockSpec((1, _R1, _K), lambda b, i: (b, i, 0)),
            pl.BlockSpec((_K, _R1), lambda b, i: (0, b * (_N // _R1) + i)),
            pl.BlockSpec((1, _R1, _H), lambda b, i: (b, i, 0)),
            pl.BlockSpec((1, _R1, _H // 2), lambda b, i: (b, i, 0)),
        ],
        out_shape=[
            jax.ShapeDtypeStruct((_B, _N, _K), jnp.int32),
            jax.ShapeDtypeStruct((_K, _B * _N), jnp.int32),
            jax.ShapeDtypeStruct((_B, _N, _H), jnp.float32),
            jax.ShapeDtypeStruct((_B, _N, _H // 2), jnp.uint32),
        ],
    )(cen, of, siz, lang, w1a, w1b, wgc, wgs, w1l, b1)


def _sc_gather(gidxt, c_all):
    n_idx = _B * _N * _K
    per_w = n_idx // _NW
    n_chunks = per_w // _GCHUNK

    mesh = plsc.VectorSubcoreMesh(core_axis_name="c", subcore_axis_name="s")

    @functools.partial(
        pl.kernel,
        mesh=mesh,
        out_type=jax.ShapeDtypeStruct((n_idx, _H // 2), jnp.uint32),
        scratch_types=[
            pltpu.VMEM((per_w,), jnp.int32),
            pltpu.VMEM((2, _GCHUNK, _H // 2), jnp.uint32),
            pltpu.SemaphoreType.DMA,
        ],
    )
    def k(gidx_hbm, c_hbm, out_hbm, idx_v, rows_v, gsem):
        wid = lax.axis_index("s") * _NC + lax.axis_index("c")
        base0 = wid * per_w
        pltpu.sync_copy(
            gidx_hbm.at[wid // _B, pl.ds((wid % _B) * per_w, per_w)], idx_v)
        cps = [None, None]
        cps[0] = pltpu.async_copy(
            c_hbm.at[idx_v.at[pl.ds(0, _GCHUNK)]], rows_v.at[0], gsem)
        for t in range(n_chunks):
            s = t % 2
            if t + 1 < n_chunks:
                cps[1 - s] = pltpu.async_copy(
                    c_hbm.at[idx_v.at[pl.ds((t + 1) * _GCHUNK, _GCHUNK)]],
                    rows_v.at[1 - s], gsem)
            cps[s].wait()
            pltpu.sync_copy(
                rows_v.at[s], out_hbm.at[pl.ds(base0 + t * _GCHUNK, _GCHUNK)])

    return k(gidxt, c_all)


def _stage3_body(a_ref, cg_ref, idx_ref, offull_ref, w2_ref, b2_ref,
                 out_ref, w_ref):
    i = pl.program_id(1)
    a = a_ref[0]
    cols = []
    for k in range(_K):
        pk = cg_ref[k]
        lo = lax.bitcast_convert_type(pk << 16, jnp.float32)
        hi = lax.bitcast_convert_type(pk & jnp.uint32(0xFFFF0000), jnp.float32)
        cgk = jnp.concatenate([lo, hi], axis=1)
        h = jnp.maximum(a + cgk, 0.0)
        cols.append(_dot_t(h, w2_ref[...]))
    scores = jnp.concatenate(cols, axis=1) + b2_ref[...]
    m = jnp.max(scores, axis=1, keepdims=True)
    e = jnp.exp(scores - m)
    w = e / jnp.sum(e, axis=1, keepdims=True)
    idx = idx_ref[0]
    colid = lax.broadcasted_iota(jnp.int32, (_R3, _N), 1)
    wmat = jnp.zeros((_R3, _N), jnp.float32)
    for k in range(_K):
        wmat = jnp.where(colid == idx[:, k:k + 1], w[:, k:k + 1], wmat)
    ctx = jnp.dot(wmat, offull_ref[0], preferred_element_type=jnp.float32)
    out_ref[0] = offull_ref[0, pl.ds(i * _R3, _R3), :] + ctx
    w_ref[0] = w


def _stage3(a_all, cg3, idx, of, w2, b2):
    grid = (_B, _N // _R3)
    nb = _B * _N // _R3
    return pl.pallas_call(
        _stage3_body,
        grid=grid,
        in_specs=[
            pl.BlockSpec((1, _R3, _H), lambda b, i: (b, i, 0)),
            pl.BlockSpec((_K, _R3, _H // 2),
                         lambda b, i: (0, b * (_N // _R3) + i, 0)),
            pl.BlockSpec((1, _R3, _K), lambda b, i: (b, i, 0)),
            pl.BlockSpec((1, _N, _D), lambda b, i: (b, 0, 0)),
            pl.BlockSpec((1, _H), lambda b, i: (0, 0)),
            pl.BlockSpec((1, _K), lambda b, i: (0, 0)),
        ],
        out_specs=[
            pl.BlockSpec((1, _R3, _D), lambda b, i: (b, i, 0)),
            pl.BlockSpec((1, _R3, _K), lambda b, i: (b, i, 0)),
        ],
        out_shape=[
            jax.ShapeDtypeStruct((_B, _N, _D), jnp.float32),
            jax.ShapeDtypeStruct((_B, _N, _K), jnp.float32),
        ],
    )(a_all, cg3, idx, of, w2, b2)


def kernel(object_features, language_embedding, centers, sizes, W1, b1, W2, b2):
    w1a = W1[:, :_D]
    w1b = W1[:, _D:2 * _D]
    wgc = W1[:, 2 * _D:2 * _D + 3]
    wgs = W1[:, 2 * _D + 3:2 * _D + 6]
    w1l = W1[:, 2 * _D + 6:]
    b1r = b1.reshape(1, _H)
    b2r = jnp.broadcast_to(b2.reshape(1, 1), (1, _K))

    idx, gidxt, a_all, c_all = _stage1(
        centers, object_features, sizes,
        language_embedding.reshape(_B, 1, _DL),
        w1a, w1b, wgc, wgs, w1l, b1r)

    cg = _sc_gather(gidxt, c_all.reshape(_B * _N, _H // 2))

    out, w = _stage3(
        a_all, cg.reshape(_K, _B * _N, _H // 2), idx, object_features, W2, b2r)

    return (out, w, idx)


# speedup vs baseline: 1.2480x; 1.0211x over previous
import functools

import jax
import jax.numpy as jnp
from jax import lax
from jax.experimental import pallas as pl
from jax.experimental.pallas import tpu as pltpu
from jax.experimental.pallas import tpu_sc as plsc

_B, _N, _D = 4, 1024, 320
_DL, _H, _K = 256, 256, 8
_R1 = 1024
_R3 = 1024
_NC, _NS = 2, 16
_NW = _NC * _NS
_GCHUNK = 128
_CW = 640


def _dot_t(x, w):
    return lax.dot_general(x, w, (((1,), (1,)), ((), ())),
                           preferred_element_type=jnp.float32)


def _stage1_body(cenall_ref, of_ref, siz_ref, lang_ref,
                 w1a_ref, w1b_ref, wgc_ref, wgs_ref, w1l_ref, b1_ref,
                 idx_ref, gidxt_ref, a_ref, c_ref):
    b = pl.program_id(0)
    i = pl.program_id(1)
    cen = cenall_ref[0, pl.ds(i * _R1, _R1), :]
    cenT = jnp.transpose(cenall_ref[0], (1, 0))
    d = jnp.zeros((_R1, _N), jnp.float32)
    for c in range(3):
        diff = cen[:, c:c + 1] - cenT[c:c + 1, :]
        d = d + diff * diff
    d = d / 25.0
    rowg = i * _R1 + lax.broadcasted_iota(jnp.int32, (_R1, _N), 0)
    colid_i = lax.broadcasted_iota(jnp.int32, (_R1, _N), 1)
    d = jnp.where(rowg == colid_i, jnp.inf, d)
    colid = colid_i.astype(jnp.float32)
    sels = []
    for k in range(_K):
        m = jnp.min(d, axis=1, keepdims=True)
        sel = jnp.min(jnp.where(d == m, colid, jnp.float32(_N)),
                      axis=1, keepdims=True)
        sels.append(sel)
        d = jnp.where(colid == sel, jnp.inf, d)
    idxf = jnp.concatenate(sels, axis=1)
    idx_ref[0] = idxf.astype(jnp.int32)
    gidxt_ref[...] = (jnp.transpose(idxf, (1, 0))
                      + jnp.float32(b * _N)).astype(jnp.int32)

    of = of_ref[0]
    g = (_dot_t(cen, wgc_ref[...]) * (1.0 / (5.0 + 1e-06))
         + _dot_t(siz_ref[0], wgs_ref[...]) * 0.5)
    langp = _dot_t(lang_ref[0], w1l_ref[...])
    a_ref[0] = _dot_t(of, w1a_ref[...]) + langp + b1_ref[...] + g
    cvals = _dot_t(of, w1b_ref[...]) - g
    u = lax.bitcast_convert_type(cvals, jnp.uint32)
    rb = (u + jnp.uint32(0x7FFF) + ((u >> 16) & jnp.uint32(1))) >> 16
    c_ref[0] = (rb[:, :_H // 2] | (rb[:, _H // 2:] << 16)).astype(jnp.uint32)


def _stage1(cen, of, siz, lang, w1a, w1b, wgc, wgs, w1l, b1):
    grid = (_B, _N // _R1)
    return pl.pallas_call(
        _stage1_body,
        grid=grid,
        in_specs=[
            pl.BlockSpec((1, _N, 3), lambda b, i: (b, 0, 0)),
            pl.BlockSpec((1, _R1, _D), lambda b, i: (b, i, 0)),
            pl.BlockSpec((1, _R1, 3), lambda b, i: (b, i, 0)),
            pl.BlockSpec((1, 1, _DL), lambda b, i: (b, 0, 0)),
            pl.BlockSpec((_H, _D), lambda b, i: (0, 0)),
            pl.BlockSpec((_H, _D), lambda b, i: (0, 0)),
            pl.BlockSpec((_H, 3), lambda b, i: (0, 0)),
            pl.BlockSpec((_H, 3), lambda b, i: (0, 0)),
            pl.BlockSpec((_H, _DL), lambda b, i: (0, 0)),
            pl.BlockSpec((1, _H), lambda b, i: (0, 0)),
        ],
        out_specs=[
            pl.BlockSpec((1, _R1, _K), lambda b, i: (b, i, 0)),
            pl.BlockSpec((_K, _R1), lambda b, i: (0, b * (_N // _R1) + i)),
            pl.BlockSpec((1, _R1, _H), lambda b, i: (b, i, 0)),
            pl.BlockSpec((1, _R1, _H // 2), lambda b, i: (b, i, 0)),
        ],
        out_shape=[
            jax.ShapeDtypeStruct((_B, _N, _K), jnp.int32),
            jax.ShapeDtypeStruct((_K, _B * _N), jnp.int32),
            jax.ShapeDtypeStruct((_B, _N, _H), jnp.float32),
            jax.ShapeDtypeStruct((_B, _N, _H // 2), jnp.uint32),
        ],
    )(cen, of, siz, lang, w1a, w1b, wgc, wgs, w1l, b1)


def _sc_gather(gidxt, c_all):
    n_idx = _B * _N * _K
    per_w = n_idx // _NW
    n_chunks = per_w // _GCHUNK

    mesh = plsc.VectorSubcoreMesh(core_axis_name="c", subcore_axis_name="s")

    @functools.partial(
        pl.kernel,
        mesh=mesh,
        out_type=jax.ShapeDtypeStruct((n_idx, _H // 2), jnp.uint32),
        scratch_types=[
            pltpu.VMEM((per_w,), jnp.int32),
            pltpu.VMEM((2, _GCHUNK, _H // 2), jnp.uint32),
            pltpu.SemaphoreType.DMA,
        ],
    )
    def k(gidx_hbm, c_hbm, out_hbm, idx_v, rows_v, gsem):
        wid = lax.axis_index("s") * _NC + lax.axis_index("c")
        base0 = wid * per_w
        pltpu.sync_copy(
            gidx_hbm.at[wid // _B, pl.ds((wid % _B) * per_w, per_w)], idx_v)
        cps = [None, None]
        cps[0] = pltpu.async_copy(
            c_hbm.at[idx_v.at[pl.ds(0, _GCHUNK)]], rows_v.at[0], gsem)
        for t in range(n_chunks):
            s = t % 2
            if t + 1 < n_chunks:
                cps[1 - s] = pltpu.async_copy(
                    c_hbm.at[idx_v.at[pl.ds((t + 1) * _GCHUNK, _GCHUNK)]],
                    rows_v.at[1 - s], gsem)
            cps[s].wait()
            pltpu.sync_copy(
                rows_v.at[s], out_hbm.at[pl.ds(base0 + t * _GCHUNK, _GCHUNK)])

    return k(gidxt, c_all)


def _stage3_body(a_ref, cg_ref, idx_ref, offull_ref, w2_ref, b2_ref,
                 out_ref, w_ref):
    i = pl.program_id(1)
    a = a_ref[0]
    cols = []
    for k in range(_K):
        pk = cg_ref[k]
        lo = lax.bitcast_convert_type(pk << 16, jnp.float32)
        hi = lax.bitcast_convert_type(pk & jnp.uint32(0xFFFF0000), jnp.float32)
        cgk = jnp.concatenate([lo, hi], axis=1)
        h = jnp.maximum(a + cgk, 0.0)
        cols.append(_dot_t(h, w2_ref[...]))
    scores = jnp.concatenate(cols, axis=1) + b2_ref[...]
    m = jnp.max(scores, axis=1, keepdims=True)
    e = jnp.exp(scores - m)
    w = e / jnp.sum(e, axis=1, keepdims=True)
    idx = idx_ref[0]
    colid = lax.broadcasted_iota(jnp.int32, (_R3, _N), 1)
    wmat = jnp.zeros((_R3, _N), jnp.float32)
    for k in range(_K):
        wmat = jnp.where(colid == idx[:, k:k + 1], w[:, k:k + 1], wmat)
    ctx = jnp.dot(wmat, offull_ref[0], preferred_element_type=jnp.float32)
    out_ref[0] = offull_ref[0, pl.ds(i * _R3, _R3), :] + ctx
    w_ref[0] = w


def _stage3(a_all, cg3, idx, of, w2, b2):
    grid = (_B, _N // _R3)
    nb = _B * _N // _R3
    return pl.pallas_call(
        _stage3_body,
        grid=grid,
        in_specs=[
            pl.BlockSpec((1, _R3, _H), lambda b, i: (b, i, 0)),
            pl.BlockSpec((_K, _R3, _H // 2),
                         lambda b, i: (0, b * (_N // _R3) + i, 0)),
            pl.BlockSpec((1, _R3, _K), lambda b, i: (b, i, 0)),
            pl.BlockSpec((1, _N, _D), lambda b, i: (b, 0, 0)),
            pl.BlockSpec((1, _H), lambda b, i: (0, 0)),
            pl.BlockSpec((1, _K), lambda b, i: (0, 0)),
        ],
        out_specs=[
            pl.BlockSpec((1, _R3, _D), lambda b, i: (b, i, 0)),
            pl.BlockSpec((1, _R3, _K), lambda b, i: (b, i, 0)),
        ],
        out_shape=[
            jax.ShapeDtypeStruct((_B, _N, _D), jnp.float32),
            jax.ShapeDtypeStruct((_B, _N, _K), jnp.float32),
        ],
    )(a_all, cg3, idx, of, w2, b2)


def kernel(object_features, language_embedding, centers, sizes, W1, b1, W2, b2):
    w1a = W1[:, :_D]
    w1b = W1[:, _D:2 * _D]
    wgc = W1[:, 2 * _D:2 * _D + 3]
    wgs = W1[:, 2 * _D + 3:2 * _D + 6]
    w1l = W1[:, 2 * _D + 6:]
    b1r = b1.reshape(1, _H)
    b2r = jnp.broadcast_to(b2.reshape(1, 1), (1, _K))

    idx, gidxt, a_all, c_all = _stage1(
        centers, object_features, sizes,
        language_embedding.reshape(_B, 1, _DL),
        w1a, w1b, wgc, wgs, w1l, b1r)

    cg = _sc_gather(gidxt, c_all.reshape(_B * _N, _H // 2))

    out, w = _stage3(
        a_all, cg.reshape(_K, _B * _N, _H // 2), idx, object_features, W2, b2r)

    return (out, w, idx)
